# baseline (speedup 1.0000x reference)
import functools

import jax
import jax.numpy as jnp
from jax import lax
from jax.experimental import pallas as pl
from jax.experimental.pallas import tpu as pltpu
from jax.experimental.pallas import tpu_sc as plsc

B = 16384
D = 32
NC, NS, L = 2, 16, 16
NW = NC * NS
BPW = B // NW
CHUNK = 128
NCHUNK = BPW // CHUNK


def _sc_dots(u, i, j, Wt, Ht):
    mesh = plsc.VectorSubcoreMesh(core_axis_name="c", subcore_axis_name="s")

    @functools.partial(
        pl.kernel,
        out_type=jax.ShapeDtypeStruct((B,), jnp.float32),
        mesh=mesh,
        scratch_types=[
            pltpu.VMEM((BPW,), jnp.int32),
            pltpu.VMEM((BPW,), jnp.int32),
            pltpu.VMEM((BPW,), jnp.int32),
            pltpu.VMEM((D, NCHUNK, CHUNK), jnp.float32),
            pltpu.VMEM((D, NCHUNK, CHUNK), jnp.float32),
            pltpu.VMEM((D, NCHUNK, CHUNK), jnp.float32),
            pltpu.VMEM((BPW,), jnp.float32),
            pltpu.SemaphoreType.DMA,
        ],
        compiler_params=pltpu.CompilerParams(
            needs_layout_passes=False, use_tc_tiling_on_sc=False),
    )
    def k(u_hbm, i_hbm, j_hbm, W_hbm, H_hbm, out_hbm,
          idx_u, idx_i, idx_j, u_buf, i_buf, j_buf, x_out, sem):
        wid = lax.axis_index("s") * NC + lax.axis_index("c")
        base = pl.multiple_of(wid * BPW, BPW)

        pltpu.sync_copy(u_hbm.at[pl.ds(base, BPW)], idx_u)
        pltpu.sync_copy(i_hbm.at[pl.ds(base, BPW)], idx_i)
        pltpu.sync_copy(j_hbm.at[pl.ds(base, BPW)], idx_j)

        def fire(d, carry):
            descs = []
            for c in range(NCHUNK):
                cs = pl.ds(c * CHUNK, CHUNK)
                descs.append(pltpu.async_copy(
                    W_hbm.at[d].at[idx_u.at[cs]], u_buf.at[d, c], sem))
                descs.append(pltpu.async_copy(
                    H_hbm.at[d].at[idx_i.at[cs]], i_buf.at[d, c], sem))
                descs.append(pltpu.async_copy(
                    H_hbm.at[d].at[idx_j.at[cs]], j_buf.at[d, c], sem))
            for dsc in descs:
                dsc.wait()
            return carry

        lax.fori_loop(0, D, fire, 0)

        for c in range(NCHUNK):
            def body(g, carry, c=c):
                gsl = pl.ds(g * L, L)
                acc_ui = jnp.zeros((L,), jnp.float32)
                acc_uj = jnp.zeros((L,), jnp.float32)
                for d in range(D):
                    uv = u_buf[d, c, gsl]
                    iv = i_buf[d, c, gsl]
                    jv = j_buf[d, c, gsl]
                    acc_ui = acc_ui + uv * iv
                    acc_uj = acc_uj + uv * jv
                x_out[pl.ds(c * CHUNK + g * L, L)] = acc_ui - acc_uj
                return carry

            lax.fori_loop(0, CHUNK // L, body, 0)

        pltpu.sync_copy(x_out, out_hbm.at[pl.ds(base, BPW)])

    return k(u, i, j, Wt, Ht)


def _neg_logsig_sum(x):
    def body(x_ref, o_ref):
        v = x_ref[...]
        sp = jnp.maximum(-v, 0.0) + jnp.log(1.0 + jnp.exp(-jnp.abs(v)))
        o_ref[0, 0] = jnp.sum(sp)

    out = pl.pallas_call(
        body,
        out_shape=jax.ShapeDtypeStruct((1, 1), jnp.float32),
        out_specs=pl.BlockSpec(memory_space=pltpu.SMEM),
    )(x.reshape(128, 128))
    return out[0, 0]


def kernel(u, i, j, W, H):
    x = _sc_dots(u.astype(jnp.int32), i.astype(jnp.int32), j.astype(jnp.int32),
                 W.T, H.T)
    return _neg_logsig_sum(x)


# speedup vs baseline: 1.0013x; 1.0013x over previous
import functools

import jax
import jax.numpy as jnp
from jax import lax
from jax.experimental import pallas as pl
from jax.experimental.pallas import tpu as pltpu
from jax.experimental.pallas import tpu_sc as plsc

B = 16384
D = 32
NC, NS, L = 2, 16, 16
NW = NC * NS
BPW = B // NW
CHUNK = 128
NCHUNK = BPW // CHUNK


def _sc_dots(u, i, j, Wt, Ht):
    mesh = plsc.VectorSubcoreMesh(core_axis_name="c", subcore_axis_name="s")

    @functools.partial(
        pl.kernel,
        out_type=jax.ShapeDtypeStruct((B,), jnp.float32),
        mesh=mesh,
        scratch_types=[
            pltpu.VMEM((BPW,), jnp.int32),
            pltpu.VMEM((BPW,), jnp.int32),
            pltpu.VMEM((BPW,), jnp.int32),
            pltpu.VMEM((D, BPW), jnp.float32),
            pltpu.VMEM((D, BPW), jnp.float32),
            pltpu.VMEM((D, BPW), jnp.float32),
            pltpu.VMEM((BPW,), jnp.float32),
            pltpu.SemaphoreType.DMA,
        ],
        compiler_params=pltpu.CompilerParams(
            needs_layout_passes=False, use_tc_tiling_on_sc=False),
    )
    def k(u_hbm, i_hbm, j_hbm, W_hbm, H_hbm, out_hbm,
          idx_u, idx_i, idx_j, u_buf, i_buf, j_buf, x_out, sem):
        wid = lax.axis_index("s") * NC + lax.axis_index("c")
        base = pl.multiple_of(wid * BPW, BPW)

        pltpu.sync_copy(u_hbm.at[pl.ds(base, BPW)], idx_u)
        pltpu.sync_copy(i_hbm.at[pl.ds(base, BPW)], idx_i)
        pltpu.sync_copy(j_hbm.at[pl.ds(base, BPW)], idx_j)

        def fire(d, carry):
            pltpu.async_copy(W_hbm.at[d].at[idx_u], u_buf.at[d], sem)
            pltpu.async_copy(H_hbm.at[d].at[idx_i], i_buf.at[d], sem)
            pltpu.async_copy(H_hbm.at[d].at[idx_j], j_buf.at[d], sem)
            return carry

        lax.fori_loop(0, D, fire, 0)

        def drain(d, carry):
            pltpu.make_async_copy(W_hbm.at[d].at[idx_u], u_buf.at[d], sem).wait()
            pltpu.make_async_copy(H_hbm.at[d].at[idx_i], i_buf.at[d], sem).wait()
            pltpu.make_async_copy(H_hbm.at[d].at[idx_j], j_buf.at[d], sem).wait()
            return carry

        lax.fori_loop(0, D, drain, 0)

        def body(g, carry):
            gsl = pl.ds(g * L, L)
            acc_ui = jnp.zeros((L,), jnp.float32)
            acc_uj = jnp.zeros((L,), jnp.float32)
            for d in range(D):
                uv = u_buf[d, gsl]
                iv = i_buf[d, gsl]
                jv = j_buf[d, gsl]
                acc_ui = acc_ui + uv * iv
                acc_uj = acc_uj + uv * jv
            x_out[gsl] = acc_ui - acc_uj
            return carry

        lax.fori_loop(0, BPW // L, body, 0)

        pltpu.sync_copy(x_out, out_hbm.at[pl.ds(base, BPW)])

    return k(u, i, j, Wt, Ht)


def _neg_logsig_sum(x):
    def body(x_ref, o_ref):
        v = x_ref[...]
        sp = jnp.maximum(-v, 0.0) + jnp.log(1.0 + jnp.exp(-jnp.abs(v)))
        o_ref[0, 0] = jnp.sum(sp)

    out = pl.pallas_call(
        body,
        out_shape=jax.ShapeDtypeStruct((1, 1), jnp.float32),
        out_specs=pl.BlockSpec(memory_space=pltpu.SMEM),
    )(x.reshape(128, 128))
    return out[0, 0]


def kernel(u, i, j, W, H):
    x = _sc_dots(u.astype(jnp.int32), i.astype(jnp.int32), j.astype(jnp.int32),
                 W.T, H.T)
    return _neg_logsig_sum(x)


# speedup vs baseline: 21.1379x; 21.1100x over previous
import functools

import jax
import jax.numpy as jnp
from jax import lax
from jax.experimental import pallas as pl
from jax.experimental.pallas import tpu as pltpu
from jax.experimental.pallas import tpu_sc as plsc

B = 16384
D = 32
ROWS = 1_000_000
CW = 32768
NBLK = (ROWS + CW - 1) // CW
BLKW = D * CW
NC, NS, L = 2, 16, 16
NW = NC * NS
BPW = B // NW


def _detile(Tt):
    def body(in_ref, o_ref):
        for d in range(D):
            o_ref[pl.ds(d * CW, CW)] = in_ref[d, :]

    return pl.pallas_call(
        body,
        grid=(NBLK,),
        in_specs=[pl.BlockSpec((D, CW), lambda k: (0, k))],
        out_specs=pl.BlockSpec((BLKW,), lambda k: (k,)),
        out_shape=jax.ShapeDtypeStruct((NBLK * BLKW,), jnp.float32),
    )(Tt)


def _sc_dots(u, i, j, Wl, Hl):
    mesh = plsc.VectorSubcoreMesh(core_axis_name="c", subcore_axis_name="s")

    @functools.partial(
        pl.kernel,
        out_type=jax.ShapeDtypeStruct((B,), jnp.float32),
        mesh=mesh,
        scratch_types=[
            pltpu.VMEM((BPW,), jnp.int32),
            pltpu.VMEM((BPW,), jnp.int32),
            pltpu.VMEM((BPW,), jnp.int32),
            pltpu.VMEM((D, BPW), jnp.int32),
            pltpu.VMEM((D, BPW), jnp.int32),
            pltpu.VMEM((D, BPW), jnp.int32),
            pltpu.VMEM((D, BPW), jnp.float32),
            pltpu.VMEM((D, BPW), jnp.float32),
            pltpu.VMEM((D, BPW), jnp.float32),
            pltpu.VMEM((BPW,), jnp.float32),
            pltpu.SemaphoreType.DMA,
        ],
        compiler_params=pltpu.CompilerParams(
            needs_layout_passes=False, use_tc_tiling_on_sc=False),
    )
    def k(u_hbm, i_hbm, j_hbm, W_hbm, H_hbm, out_hbm,
          idx_u, idx_i, idx_j, fidx_u, fidx_i, fidx_j,
          u_buf, i_buf, j_buf, x_out, sem):
        wid = lax.axis_index("s") * NC + lax.axis_index("c")
        base = pl.multiple_of(wid * BPW, BPW)

        pltpu.sync_copy(u_hbm.at[pl.ds(base, BPW)], idx_u)
        pltpu.sync_copy(i_hbm.at[pl.ds(base, BPW)], idx_i)
        pltpu.sync_copy(j_hbm.at[pl.ds(base, BPW)], idx_j)

        def addr(g, carry):
            gsl = pl.ds(g * L, L)
            for idx_r, fidx_r in ((idx_u, fidx_u), (idx_i, fidx_i),
                                  (idx_j, fidx_j)):
                v = idx_r[gsl]
                pre = lax.shift_left(lax.shift_right_logical(v, 15), 20) + \
                    jnp.bitwise_and(v, CW - 1)
                for d in range(D):
                    fidx_r[d, gsl] = pre + (d * CW)
            return carry

        lax.fori_loop(0, BPW // L, addr, 0)

        def fire(d, carry):
            pltpu.async_copy(W_hbm.at[fidx_u.at[d]], u_buf.at[d], sem)
            pltpu.async_copy(H_hbm.at[fidx_i.at[d]], i_buf.at[d], sem)
            pltpu.async_copy(H_hbm.at[fidx_j.at[d]], j_buf.at[d], sem)
            return carry

        lax.fori_loop(0, D, fire, 0)

        def drain(d, carry):
            pltpu.make_async_copy(W_hbm.at[fidx_u.at[d]], u_buf.at[d], sem).wait()
            pltpu.make_async_copy(H_hbm.at[fidx_i.at[d]], i_buf.at[d], sem).wait()
            pltpu.make_async_copy(H_hbm.at[fidx_j.at[d]], j_buf.at[d], sem).wait()
            return carry

        lax.fori_loop(0, D, drain, 0)

        def body(g, carry):
            gsl = pl.ds(g * L, L)
            acc_ui = jnp.zeros((L,), jnp.float32)
            acc_uj = jnp.zeros((L,), jnp.float32)
            for d in range(D):
                uv = u_buf[d, gsl]
                iv = i_buf[d, gsl]
                jv = j_buf[d, gsl]
                acc_ui = acc_ui + uv * iv
                acc_uj = acc_uj + uv * jv
            x_out[gsl] = acc_ui - acc_uj
            return carry

        lax.fori_loop(0, BPW // L, body, 0)

        pltpu.sync_copy(x_out, out_hbm.at[pl.ds(base, BPW)])

    return k(u, i, j, Wl, Hl)


def _neg_logsig_sum(x):
    def body(x_ref, o_ref):
        v = x_ref[...]
        sp = jnp.maximum(-v, 0.0) + jnp.log(1.0 + jnp.exp(-jnp.abs(v)))
        o_ref[0, 0] = jnp.sum(sp)

    out = pl.pallas_call(
        body,
        out_shape=jax.ShapeDtypeStruct((1, 1), jnp.float32),
        out_specs=pl.BlockSpec(memory_space=pltpu.SMEM),
    )(x.reshape(128, 128))
    return out[0, 0]


def kernel(u, i, j, W, H):
    Wl = _detile(W.T)
    Hl = _detile(H.T)
    x = _sc_dots(u.astype(jnp.int32), i.astype(jnp.int32), j.astype(jnp.int32),
                 Wl, Hl)
    return _neg_logsig_sum(x)


# speedup vs baseline: 21.1876x; 1.0023x over previous
import functools

import jax
import jax.numpy as jnp
from jax import lax
from jax.experimental import pallas as pl
from jax.experimental.pallas import tpu as pltpu
from jax.experimental.pallas import tpu_sc as plsc

B = 16384
D = 32
ROWS = 1_000_000
CW = 32768
NBLK = (ROWS + CW - 1) // CW
BLKW = D * CW
NC, NS, L = 2, 16, 16
NW = NC * NS
BPW = B // NW


def _detile(Tt):
    def body(in_ref, o_ref):
        for d in range(D):
            o_ref[pl.ds(d * CW, CW)] = in_ref[d, :]

    return pl.pallas_call(
        body,
        grid=(NBLK,),
        in_specs=[pl.BlockSpec((D, CW), lambda k: (0, k))],
        out_specs=pl.BlockSpec((BLKW,), lambda k: (k,)),
        out_shape=jax.ShapeDtypeStruct((NBLK * BLKW,), jnp.float32),
    )(Tt)


def _sc_dots(u, i, j, Wl, Hl):
    mesh = plsc.VectorSubcoreMesh(core_axis_name="c", subcore_axis_name="s")

    @functools.partial(
        pl.kernel,
        out_type=jax.ShapeDtypeStruct((B,), jnp.float32),
        mesh=mesh,
        scratch_types=[
            pltpu.VMEM((BPW,), jnp.int32),
            pltpu.VMEM((BPW,), jnp.int32),
            pltpu.VMEM((BPW,), jnp.int32),
            pltpu.VMEM((D, BPW), jnp.int32),
            pltpu.VMEM((D, BPW), jnp.int32),
            pltpu.VMEM((D, BPW), jnp.int32),
            pltpu.VMEM((D, BPW), jnp.float32),
            pltpu.VMEM((D, BPW), jnp.float32),
            pltpu.VMEM((D, BPW), jnp.float32),
            pltpu.VMEM((BPW,), jnp.float32),
            pltpu.SemaphoreType.DMA,
        ],
        compiler_params=pltpu.CompilerParams(
            needs_layout_passes=False, use_tc_tiling_on_sc=False),
    )
    def k(u_hbm, i_hbm, j_hbm, W_hbm, H_hbm, out_hbm,
          idx_u, idx_i, idx_j, fidx_u, fidx_i, fidx_j,
          u_buf, i_buf, j_buf, x_out, sem):
        wid = lax.axis_index("s") * NC + lax.axis_index("c")
        base = pl.multiple_of(wid * BPW, BPW)

        pltpu.sync_copy(u_hbm.at[pl.ds(base, BPW)], idx_u)
        pltpu.sync_copy(i_hbm.at[pl.ds(base, BPW)], idx_i)
        pltpu.sync_copy(j_hbm.at[pl.ds(base, BPW)], idx_j)

        def fire(d, carry):
            def addr(g, carry2):
                gsl = pl.ds(g * L, L)
                for idx_r, fidx_r in ((idx_u, fidx_u), (idx_i, fidx_i),
                                      (idx_j, fidx_j)):
                    v = idx_r[gsl]
                    pre = lax.shift_left(lax.shift_right_logical(v, 15), 20) \
                        + jnp.bitwise_and(v, CW - 1)
                    fidx_r[d, gsl] = pre + d * CW
                return carry2

            lax.fori_loop(0, BPW // L, addr, 0)
            pltpu.async_copy(W_hbm.at[fidx_u.at[d]], u_buf.at[d], sem)
            pltpu.async_copy(H_hbm.at[fidx_i.at[d]], i_buf.at[d], sem)
            pltpu.async_copy(H_hbm.at[fidx_j.at[d]], j_buf.at[d], sem)
            return carry

        lax.fori_loop(0, D, fire, 0)

        def drain(d, carry):
            pltpu.make_async_copy(W_hbm.at[fidx_u.at[d]], u_buf.at[d], sem).wait()
            pltpu.make_async_copy(H_hbm.at[fidx_i.at[d]], i_buf.at[d], sem).wait()
            pltpu.make_async_copy(H_hbm.at[fidx_j.at[d]], j_buf.at[d], sem).wait()
            return carry

        lax.fori_loop(0, D, drain, 0)

        def body(g, carry):
            gsl = pl.ds(g * L, L)
            acc_ui = jnp.zeros((L,), jnp.float32)
            acc_uj = jnp.zeros((L,), jnp.float32)
            for d in range(D):
                uv = u_buf[d, gsl]
                iv = i_buf[d, gsl]
                jv = j_buf[d, gsl]
                acc_ui = acc_ui + uv * iv
                acc_uj = acc_uj + uv * jv
            x_out[gsl] = acc_ui - acc_uj
            return carry

        lax.fori_loop(0, BPW // L, body, 0)

        pltpu.sync_copy(x_out, out_hbm.at[pl.ds(base, BPW)])

    return k(u, i, j, Wl, Hl)


def _neg_logsig_sum(x):
    def body(x_ref, o_ref):
        v = x_ref[...]
        sp = jnp.maximum(-v, 0.0) + jnp.log(1.0 + jnp.exp(-jnp.abs(v)))
        o_ref[0, 0] = jnp.sum(sp)

    out = pl.pallas_call(
        body,
        out_shape=jax.ShapeDtypeStruct((1, 1), jnp.float32),
        out_specs=pl.BlockSpec(memory_space=pltpu.SMEM),
    )(x.reshape(128, 128))
    return out[0, 0]


def kernel(u, i, j, W, H):
    Wl = _detile(W.T)
    Hl = _detile(H.T)
    x = _sc_dots(u.astype(jnp.int32), i.astype(jnp.int32), j.astype(jnp.int32),
                 Wl, Hl)
    return _neg_logsig_sum(x)
